# baseline (speedup 1.0000x reference)
import functools

import jax
import jax.numpy as jnp
from jax import lax
from jax.experimental import pallas as pl
from jax.experimental.pallas import tpu as pltpu
from jax.experimental.pallas import tpu_sc as plsc

B = 16384
D = 64
NC = 2
NS = 16
NW = NC * NS
BPW = B // NW
CH = 128
NCH = BPW // CH
WAVE = 16
NWAVE = BPW // WAVE


def _wave_indices(idx_s, idx_t, w):
    row = (w * WAVE) // CH
    col = (w * WAVE) % CH
    vec_s = idx_s[row, pl.ds(col, WAVE)]
    vec_t = idx_t[row, pl.ds(col, WAVE)]
    return vec_s, vec_t


def _fire_wave(emb_hbm, idx_s, idx_t, bufS, bufT, semS, semT, w):
    vec_s, vec_t = _wave_indices(idx_s, idx_t, w)
    for b in range(WAVE):
        j_s = pl.multiple_of((vec_s[b] >> 4) << 4, 16)
        j_t = pl.multiple_of((vec_t[b] >> 4) << 4, 16)
        pltpu.async_copy(emb_hbm.at[:, pl.ds(j_s, 16)], bufS.at[b], semS)
        pltpu.async_copy(emb_hbm.at[:, pl.ds(j_t, 16)], bufT.at[b], semT)


def _drain_wave(emb_hbm, bufS, bufT, semS, semT):
    for b in range(WAVE):
        pltpu.make_async_copy(
            emb_hbm.at[:, pl.ds(0, 16)], bufS.at[b], semS).wait()
        pltpu.make_async_copy(
            emb_hbm.at[:, pl.ds(0, 16)], bufT.at[b], semT).wait()


def _compute_wave(idx_s, idx_t, bufS, bufT, out_v, w):
    lane = lax.iota(jnp.int32, 16)
    res = jnp.zeros((16,), jnp.float32)
    vec_s, vec_t = _wave_indices(idx_s, idx_t, w)
    lane_s = vec_s & 15
    lane_t = vec_t & 15
    for b in range(WAVE):
        col_s = jnp.full((16,), lane_s[b], jnp.int32)
        col_t = jnp.full((16,), lane_t[b], jnp.int32)
        sref = bufS.at[b]
        tref = bufT.at[b]
        acc = jnp.zeros((16,), jnp.float32)
        for c in range(D // 16):
            rows = 16 * c + lane
            s = plsc.load_gather(sref, [rows, col_s])
            t = plsc.load_gather(tref, [rows, col_t])
            acc = acc + s * t
        res = jnp.where(lane == b, jnp.sum(acc), res)
    out_v[pl.ds(w * WAVE, WAVE)] = res


def _sc_body(src_hbm, trg_hbm, emb_hbm, out_hbm,
             idx_s, idx_t, bS0, bT0, bS1, bT1, out_v,
             sS0, sT0, sS1, sT1):
    wid = lax.axis_index("s") * NC + lax.axis_index("c")
    base_row = wid * NCH

    pltpu.sync_copy(src_hbm.at[pl.ds(base_row, NCH)], idx_s)
    pltpu.sync_copy(trg_hbm.at[pl.ds(base_row, NCH)], idx_t)

    _fire_wave(emb_hbm, idx_s, idx_t, bS0, bT0, sS0, sT0, 0)

    def body(k, carry):
        w0 = 2 * k
        w1 = 2 * k + 1
        _fire_wave(emb_hbm, idx_s, idx_t, bS1, bT1, sS1, sT1, w1)
        _drain_wave(emb_hbm, bS0, bT0, sS0, sT0)
        _compute_wave(idx_s, idx_t, bS0, bT0, out_v, w0)

        @pl.when(k < NWAVE // 2 - 1)
        def _():
            _fire_wave(emb_hbm, idx_s, idx_t, bS0, bT0, sS0, sT0, w0 + 2)

        _drain_wave(emb_hbm, bS1, bT1, sS1, sT1)
        _compute_wave(idx_s, idx_t, bS1, bT1, out_v, w1)
        return carry

    lax.fori_loop(0, NWAVE // 2, body, 0)

    pltpu.sync_copy(out_v, out_hbm.at[pl.ds(wid * BPW, BPW)])


_sc_kernel = functools.partial(
    pl.kernel,
    out_type=jax.ShapeDtypeStruct((B,), jnp.float32),
    mesh=plsc.VectorSubcoreMesh(core_axis_name="c", subcore_axis_name="s"),
    compiler_params=pltpu.CompilerParams(
        needs_layout_passes=False, use_tc_tiling_on_sc=False),
    scratch_types=[
        pltpu.VMEM((NCH, CH), jnp.int32),
        pltpu.VMEM((NCH, CH), jnp.int32),
        pltpu.VMEM((WAVE, D, 16), jnp.float32),
        pltpu.VMEM((WAVE, D, 16), jnp.float32),
        pltpu.VMEM((WAVE, D, 16), jnp.float32),
        pltpu.VMEM((WAVE, D, 16), jnp.float32),
        pltpu.VMEM((BPW,), jnp.float32),
        pltpu.SemaphoreType.DMA,
        pltpu.SemaphoreType.DMA,
        pltpu.SemaphoreType.DMA,
        pltpu.SemaphoreType.DMA,
    ],
)(_sc_body)


def kernel(network, src, trg, emb):
    src32 = src.astype(jnp.int32).reshape(NW * NCH, CH)
    trg32 = trg.astype(jnp.int32).reshape(NW * NCH, CH)
    return _sc_kernel(src32, trg32, emb.T)


# speedup vs baseline: 16.8990x; 16.8990x over previous
import functools

import jax
import jax.numpy as jnp
from jax import lax
from jax.experimental import pallas as pl
from jax.experimental.pallas import tpu as pltpu
from jax.experimental.pallas import tpu_sc as plsc

V = 1000000
B = 16384
D = 64
DP = 128
NC = 2
NS = 16
NW = NC * NS
BPW = B // NW
CH = 128
NCH = BPW // CH

IB = 8192


def _detile_body(in_ref, out_ref):
    x = in_ref[...]
    xt = jnp.transpose(x)
    out_ref[...] = jnp.concatenate(
        [xt, jnp.zeros((IB, DP - D), jnp.float32)], axis=1)


def _detile(emb_t):
    grid = (V + IB - 1) // IB
    return pl.pallas_call(
        _detile_body,
        grid=(grid,),
        in_specs=[pl.BlockSpec((D, IB), lambda i: (0, i))],
        out_specs=pl.BlockSpec((IB, DP), lambda i: (i, 0)),
        out_shape=jax.ShapeDtypeStruct((V, DP), jnp.float32),
    )(emb_t)


def _compute_chunk(rows_s, rows_t, out_v, j):
    lane = lax.iota(jnp.int32, 16)
    for g in range(CH // 16):
        res = jnp.zeros((16,), jnp.float32)
        for b in range(16):
            p = g * 16 + b
            acc = jnp.zeros((16,), jnp.float32)
            for c in range(D // 16):
                s = rows_s[p, pl.ds(c * 16, 16)]
                t = rows_t[p, pl.ds(c * 16, 16)]
                acc = acc + s * t
            res = jnp.where(lane == b, jnp.sum(acc), res)
        out_v[pl.ds(j * CH + g * 16, 16)] = res


def _sc_body(src_hbm, trg_hbm, emb_hbm, out_hbm,
             idx_s, idx_t, bS0, bT0, bS1, bT1, out_v, sem):
    wid = lax.axis_index("s") * NC + lax.axis_index("c")
    base_row = wid * NCH

    pltpu.sync_copy(src_hbm.at[pl.ds(base_row, NCH)], idx_s)
    pltpu.sync_copy(trg_hbm.at[pl.ds(base_row, NCH)], idx_t)

    def fire(j, bufS, bufT):
        pltpu.async_copy(emb_hbm.at[idx_s.at[j]], bufS, sem)
        pltpu.async_copy(emb_hbm.at[idx_t.at[j]], bufT, sem)

    def drain(bufS, bufT):
        pltpu.make_async_copy(emb_hbm.at[idx_s.at[0]], bufS, sem).wait()
        pltpu.make_async_copy(emb_hbm.at[idx_t.at[0]], bufT, sem).wait()

    fire(0, bS0, bT0)
    fire(1, bS1, bT1)
    drain(bS0, bT0)
    _compute_chunk(bS0, bT0, out_v, 0)
    fire(2, bS0, bT0)
    drain(bS1, bT1)
    _compute_chunk(bS1, bT1, out_v, 1)
    fire(3, bS1, bT1)
    drain(bS0, bT0)
    _compute_chunk(bS0, bT0, out_v, 2)
    drain(bS1, bT1)
    _compute_chunk(bS1, bT1, out_v, 3)

    pltpu.sync_copy(out_v, out_hbm.at[pl.ds(wid * BPW, BPW)])


_sc_kernel = functools.partial(
    pl.kernel,
    out_type=jax.ShapeDtypeStruct((B,), jnp.float32),
    mesh=plsc.VectorSubcoreMesh(core_axis_name="c", subcore_axis_name="s"),
    compiler_params=pltpu.CompilerParams(
        needs_layout_passes=False, use_tc_tiling_on_sc=True),
    scratch_types=[
        pltpu.VMEM((NCH, CH), jnp.int32),
        pltpu.VMEM((NCH, CH), jnp.int32),
        pltpu.VMEM((CH, DP), jnp.float32),
        pltpu.VMEM((CH, DP), jnp.float32),
        pltpu.VMEM((CH, DP), jnp.float32),
        pltpu.VMEM((CH, DP), jnp.float32),
        pltpu.VMEM((BPW,), jnp.float32),
        pltpu.SemaphoreType.DMA,
    ],
)(_sc_body)


def kernel(network, src, trg, emb):
    src32 = src.astype(jnp.int32).reshape(NW * NCH, CH)
    trg32 = trg.astype(jnp.int32).reshape(NW * NCH, CH)
    padded = _detile(emb.T)
    return _sc_kernel(src32, trg32, padded)


# speedup vs baseline: 17.5065x; 1.0359x over previous
import functools

import jax
import jax.numpy as jnp
from jax import lax
from jax.experimental import pallas as pl
from jax.experimental.pallas import tpu as pltpu
from jax.experimental.pallas import tpu_sc as plsc

V = 1000000
B = 16384
D = 64
DP = 128
NC = 2
NS = 16
NW = NC * NS
BPW = B // NW
CH = 128
NCH = BPW // CH

IB = 4096
NBLK = (V + IB - 1) // IB
NPAIR = (NBLK + 1) // 2
VP = NPAIR * IB


def _detile_body(lo_ref, hi_ref, out_ref):
    out_ref[:, 0:D] = jnp.transpose(lo_ref[...])
    out_ref[:, D:DP] = jnp.transpose(hi_ref[...])


def _detile(emb_t):
    last = NBLK - 1
    return pl.pallas_call(
        _detile_body,
        grid=(NPAIR,),
        in_specs=[
            pl.BlockSpec((D, IB), lambda i: (0, jnp.minimum(2 * i, last))),
            pl.BlockSpec((D, IB),
                         lambda i: (0, jnp.minimum(2 * i + 1, last))),
        ],
        out_specs=pl.BlockSpec((IB, DP), lambda i: (i, 0)),
        out_shape=jax.ShapeDtypeStruct((VP, DP), jnp.float32),
    )(emb_t, emb_t)


def _compute_chunk(po_s, po_t, rows_s, rows_t, out_v, j):
    lane = lax.iota(jnp.int32, 16)
    for g in range(CH // 16):
        res = jnp.zeros((16,), jnp.float32)
        ps = po_s[j, pl.ds(g * 16, 16)]
        pt = po_t[j, pl.ds(g * 16, 16)]
        for b in range(16):
            p = g * 16 + b
            off_s = pl.multiple_of(ps[b], 64)
            off_t = pl.multiple_of(pt[b], 64)
            acc = jnp.zeros((16,), jnp.float32)
            for c in range(D // 16):
                s = rows_s[p, pl.ds(off_s + c * 16, 16)]
                t = rows_t[p, pl.ds(off_t + c * 16, 16)]
                acc = acc + s * t
            res = jnp.where(lane == b, jnp.sum(acc), res)
        out_v[pl.ds(j * CH + g * 16, 16)] = res


def _sc_body(srch_hbm, trgh_hbm, spar_hbm, tpar_hbm, emb_hbm, out_hbm,
             idx_s, idx_t, po_s, po_t, bS0, bT0, bS1, bT1, out_v, sem):
    wid = lax.axis_index("s") * NC + lax.axis_index("c")
    base_row = wid * NCH

    pltpu.sync_copy(srch_hbm.at[pl.ds(base_row, NCH)], idx_s)
    pltpu.sync_copy(trgh_hbm.at[pl.ds(base_row, NCH)], idx_t)
    pltpu.sync_copy(spar_hbm.at[pl.ds(base_row, NCH)], po_s)
    pltpu.sync_copy(tpar_hbm.at[pl.ds(base_row, NCH)], po_t)

    def fire(j, bufS, bufT):
        pltpu.async_copy(emb_hbm.at[idx_s.at[j]], bufS, sem)
        pltpu.async_copy(emb_hbm.at[idx_t.at[j]], bufT, sem)

    def drain(bufS, bufT):
        pltpu.make_async_copy(emb_hbm.at[idx_s.at[0]], bufS, sem).wait()
        pltpu.make_async_copy(emb_hbm.at[idx_t.at[0]], bufT, sem).wait()

    fire(0, bS0, bT0)
    fire(1, bS1, bT1)
    drain(bS0, bT0)
    _compute_chunk(po_s, po_t, bS0, bT0, out_v, 0)
    fire(2, bS0, bT0)
    drain(bS1, bT1)
    _compute_chunk(po_s, po_t, bS1, bT1, out_v, 1)
    fire(3, bS1, bT1)
    drain(bS0, bT0)
    _compute_chunk(po_s, po_t, bS0, bT0, out_v, 2)
    drain(bS1, bT1)
    _compute_chunk(po_s, po_t, bS1, bT1, out_v, 3)

    pltpu.sync_copy(out_v, out_hbm.at[pl.ds(wid * BPW, BPW)])


_sc_kernel = functools.partial(
    pl.kernel,
    out_type=jax.ShapeDtypeStruct((B,), jnp.float32),
    mesh=plsc.VectorSubcoreMesh(core_axis_name="c", subcore_axis_name="s"),
    compiler_params=pltpu.CompilerParams(
        needs_layout_passes=False, use_tc_tiling_on_sc=True),
    scratch_types=[
        pltpu.VMEM((NCH, CH), jnp.int32),
        pltpu.VMEM((NCH, CH), jnp.int32),
        pltpu.VMEM((NCH, CH), jnp.int32),
        pltpu.VMEM((NCH, CH), jnp.int32),
        pltpu.VMEM((CH, DP), jnp.float32),
        pltpu.VMEM((CH, DP), jnp.float32),
        pltpu.VMEM((CH, DP), jnp.float32),
        pltpu.VMEM((CH, DP), jnp.float32),
        pltpu.VMEM((BPW,), jnp.float32),
        pltpu.SemaphoreType.DMA,
    ],
)(_sc_body)


def kernel(network, src, trg, emb):
    full_cols = (NBLK - 1) * IB

    def packed_row(x):
        blk = x >> 12
        j_main = ((blk >> 1) << 12) | (x & (IB - 1))
        j_tail = (NPAIR - 1) * IB + (x - full_cols)
        return jnp.where(x < full_cols, j_main, j_tail)

    def lane_off(x):
        return jnp.where(x < full_cols, ((x >> 12) & 1) << 6, 0)

    src32 = src.astype(jnp.int32)
    trg32 = trg.astype(jnp.int32)
    srch = packed_row(src32).reshape(NW * NCH, CH)
    trgh = packed_row(trg32).reshape(NW * NCH, CH)
    spar = lane_off(src32).reshape(NW * NCH, CH)
    tpar = lane_off(trg32).reshape(NW * NCH, CH)
    packed = _detile(emb.T)
    return _sc_kernel(srch, trgh, spar, tpar, packed)


# speedup vs baseline: 19.6676x; 1.1234x over previous
import functools

import jax
import jax.numpy as jnp
from jax import lax
from jax.experimental import pallas as pl
from jax.experimental.pallas import tpu as pltpu
from jax.experimental.pallas import tpu_sc as plsc

V = 1000000
B = 16384
D = 64
DP = 128
NC = 2
NS = 16
NW = NC * NS
BPW = B // NW
CH = 128
NCH = BPW // CH

IB = 8192
IBL = 13
NBLK = (V + IB - 1) // IB
NPAIR = (NBLK + 1) // 2
VP = NPAIR * IB


def _detile_body(lo_ref, hi_ref, out_ref):
    out_ref[...] = jnp.concatenate(
        [jnp.transpose(lo_ref[...]), jnp.transpose(hi_ref[...])], axis=1)


def _detile(emb_t):
    last = NBLK - 1
    return pl.pallas_call(
        _detile_body,
        grid=(NPAIR,),
        in_specs=[
            pl.BlockSpec((D, IB), lambda i: (0, jnp.minimum(2 * i, last))),
            pl.BlockSpec((D, IB),
                         lambda i: (0, jnp.minimum(2 * i + 1, last))),
        ],
        out_specs=pl.BlockSpec((IB, DP), lambda i: (i, 0)),
        out_shape=jax.ShapeDtypeStruct((VP, DP), jnp.float32),
    )(emb_t, emb_t)


def _compute_chunk(po_s, po_t, rows_s, rows_t, out_v, j):
    lane = lax.iota(jnp.int32, 16)
    for g in range(CH // 16):
        res = jnp.zeros((16,), jnp.float32)
        ps = po_s[j, pl.ds(g * 16, 16)]
        pt = po_t[j, pl.ds(g * 16, 16)]
        for b in range(16):
            p = g * 16 + b
            off_s = pl.multiple_of(ps[b], 64)
            off_t = pl.multiple_of(pt[b], 64)
            acc = jnp.zeros((16,), jnp.float32)
            for c in range(D // 16):
                s = rows_s[p, pl.ds(off_s + c * 16, 16)]
                t = rows_t[p, pl.ds(off_t + c * 16, 16)]
                acc = acc + s * t
            res = jnp.where(lane == b, jnp.sum(acc), res)
        out_v[pl.ds(j * CH + g * 16, 16)] = res


def _sc_body(srch_hbm, trgh_hbm, spar_hbm, tpar_hbm, emb_hbm, out_hbm,
             idx_s, idx_t, po_s, po_t, bS0, bT0, bS1, bT1, out_v, sem):
    wid = lax.axis_index("s") * NC + lax.axis_index("c")
    base_row = wid * NCH

    pltpu.sync_copy(srch_hbm.at[pl.ds(base_row, NCH)], idx_s)
    pltpu.sync_copy(trgh_hbm.at[pl.ds(base_row, NCH)], idx_t)
    pltpu.sync_copy(spar_hbm.at[pl.ds(base_row, NCH)], po_s)
    pltpu.sync_copy(tpar_hbm.at[pl.ds(base_row, NCH)], po_t)

    def fire(j, bufS, bufT):
        pltpu.async_copy(emb_hbm.at[idx_s.at[j]], bufS, sem)
        pltpu.async_copy(emb_hbm.at[idx_t.at[j]], bufT, sem)

    def drain(bufS, bufT):
        pltpu.make_async_copy(emb_hbm.at[idx_s.at[0]], bufS, sem).wait()
        pltpu.make_async_copy(emb_hbm.at[idx_t.at[0]], bufT, sem).wait()

    fire(0, bS0, bT0)
    fire(1, bS1, bT1)
    drain(bS0, bT0)
    _compute_chunk(po_s, po_t, bS0, bT0, out_v, 0)
    fire(2, bS0, bT0)
    drain(bS1, bT1)
    _compute_chunk(po_s, po_t, bS1, bT1, out_v, 1)
    fire(3, bS1, bT1)
    drain(bS0, bT0)
    _compute_chunk(po_s, po_t, bS0, bT0, out_v, 2)
    drain(bS1, bT1)
    _compute_chunk(po_s, po_t, bS1, bT1, out_v, 3)

    pltpu.sync_copy(out_v, out_hbm.at[pl.ds(wid * BPW, BPW)])


_sc_kernel = functools.partial(
    pl.kernel,
    out_type=jax.ShapeDtypeStruct((B,), jnp.float32),
    mesh=plsc.VectorSubcoreMesh(core_axis_name="c", subcore_axis_name="s"),
    compiler_params=pltpu.CompilerParams(
        needs_layout_passes=False, use_tc_tiling_on_sc=True),
    scratch_types=[
        pltpu.VMEM((NCH, CH), jnp.int32),
        pltpu.VMEM((NCH, CH), jnp.int32),
        pltpu.VMEM((NCH, CH), jnp.int32),
        pltpu.VMEM((NCH, CH), jnp.int32),
        pltpu.VMEM((CH, DP), jnp.float32),
        pltpu.VMEM((CH, DP), jnp.float32),
        pltpu.VMEM((CH, DP), jnp.float32),
        pltpu.VMEM((CH, DP), jnp.float32),
        pltpu.VMEM((BPW,), jnp.float32),
        pltpu.SemaphoreType.DMA,
    ],
)(_sc_body)


def kernel(network, src, trg, emb):
    full_cols = (NBLK - 1) * IB

    def packed_row(x):
        blk = x >> IBL
        j_main = ((blk >> 1) << IBL) | (x & (IB - 1))
        j_tail = (NPAIR - 1) * IB + (x - full_cols)
        return jnp.where(x < full_cols, j_main, j_tail)

    def lane_off(x):
        return jnp.where(x < full_cols, ((x >> IBL) & 1) << 6, 0)

    src32 = src.astype(jnp.int32)
    trg32 = trg.astype(jnp.int32)
    srch = packed_row(src32).reshape(NW * NCH, CH)
    trgh = packed_row(trg32).reshape(NW * NCH, CH)
    spar = lane_off(src32).reshape(NW * NCH, CH)
    tpar = lane_off(trg32).reshape(NW * NCH, CH)
    packed = _detile(emb.T)
    return _sc_kernel(srch, trgh, spar, tpar, packed)


# speedup vs baseline: 25.5066x; 1.2969x over previous
import functools

import jax
import jax.numpy as jnp
from jax import lax
from jax.experimental import pallas as pl
from jax.experimental.pallas import tpu as pltpu
from jax.experimental.pallas import tpu_sc as plsc

V = 1000000
B = 16384
D = 64
DP = 128
NC = 2
NS = 16
NW = NC * NS
BPW = B // NW
CH = 128
NCH = BPW // CH

IB = 8192
IBL = 13
NBLK = (V + IB - 1) // IB
NPAIR = (NBLK + 1) // 2
VP = NPAIR * IB


def _detile_body(in_ref, out_ref):
    x = in_ref[...]
    xc = jnp.concatenate([x[:, :IB], x[:, IB:]], axis=0)
    eye = (lax.broadcasted_iota(jnp.int32, (DP, DP), 0)
           == lax.broadcasted_iota(jnp.int32, (DP, DP), 1)
           ).astype(jnp.float32)
    out_ref[...] = lax.dot_general(
        xc, eye, (((0,), (0,)), ((), ())),
        preferred_element_type=jnp.float32)


def _detile(emb_t):
    return pl.pallas_call(
        _detile_body,
        grid=(NPAIR,),
        compiler_params=pltpu.CompilerParams(
            fuse_transposed_lhs_in_matmul=True),
        in_specs=[pl.BlockSpec((D, 2 * IB), lambda i: (0, i))],
        out_specs=pl.BlockSpec((IB, DP), lambda i: (i, 0)),
        out_shape=jax.ShapeDtypeStruct((VP, DP), jnp.float32),
    )(emb_t)


def _compute_chunk(po_s, po_t, rows_s, rows_t, out_v, j):
    lane = lax.iota(jnp.int32, 16)
    for g in range(CH // 16):
        res = jnp.zeros((16,), jnp.float32)
        ps = po_s[j, pl.ds(g * 16, 16)]
        pt = po_t[j, pl.ds(g * 16, 16)]
        for b in range(16):
            p = g * 16 + b
            off_s = pl.multiple_of(ps[b], 64)
            off_t = pl.multiple_of(pt[b], 64)
            acc = jnp.zeros((16,), jnp.float32)
            for c in range(D // 16):
                s = rows_s[p, pl.ds(off_s + c * 16, 16)]
                t = rows_t[p, pl.ds(off_t + c * 16, 16)]
                acc = acc + s * t
            res = jnp.where(lane == b, jnp.sum(acc), res)
        out_v[pl.ds(j * CH + g * 16, 16)] = res


def _sc_body(srch_hbm, trgh_hbm, spar_hbm, tpar_hbm, emb_hbm, out_hbm,
             idx_s, idx_t, po_s, po_t, bS0, bT0, bS1, bT1, out_v, sem):
    wid = lax.axis_index("s") * NC + lax.axis_index("c")
    base_row = wid * NCH

    pltpu.sync_copy(srch_hbm.at[pl.ds(base_row, NCH)], idx_s)
    pltpu.sync_copy(trgh_hbm.at[pl.ds(base_row, NCH)], idx_t)
    pltpu.sync_copy(spar_hbm.at[pl.ds(base_row, NCH)], po_s)
    pltpu.sync_copy(tpar_hbm.at[pl.ds(base_row, NCH)], po_t)

    def fire(j, bufS, bufT):
        pltpu.async_copy(emb_hbm.at[idx_s.at[j]], bufS, sem)
        pltpu.async_copy(emb_hbm.at[idx_t.at[j]], bufT, sem)

    def drain(bufS, bufT):
        pltpu.make_async_copy(emb_hbm.at[idx_s.at[0]], bufS, sem).wait()
        pltpu.make_async_copy(emb_hbm.at[idx_t.at[0]], bufT, sem).wait()

    fire(0, bS0, bT0)
    fire(1, bS1, bT1)
    drain(bS0, bT0)
    _compute_chunk(po_s, po_t, bS0, bT0, out_v, 0)
    fire(2, bS0, bT0)
    drain(bS1, bT1)
    _compute_chunk(po_s, po_t, bS1, bT1, out_v, 1)
    fire(3, bS1, bT1)
    drain(bS0, bT0)
    _compute_chunk(po_s, po_t, bS0, bT0, out_v, 2)
    drain(bS1, bT1)
    _compute_chunk(po_s, po_t, bS1, bT1, out_v, 3)

    pltpu.sync_copy(out_v, out_hbm.at[pl.ds(wid * BPW, BPW)])


_sc_kernel = functools.partial(
    pl.kernel,
    out_type=jax.ShapeDtypeStruct((B,), jnp.float32),
    mesh=plsc.VectorSubcoreMesh(core_axis_name="c", subcore_axis_name="s"),
    compiler_params=pltpu.CompilerParams(
        needs_layout_passes=False, use_tc_tiling_on_sc=True),
    scratch_types=[
        pltpu.VMEM((NCH, CH), jnp.int32),
        pltpu.VMEM((NCH, CH), jnp.int32),
        pltpu.VMEM((NCH, CH), jnp.int32),
        pltpu.VMEM((NCH, CH), jnp.int32),
        pltpu.VMEM((CH, DP), jnp.float32),
        pltpu.VMEM((CH, DP), jnp.float32),
        pltpu.VMEM((CH, DP), jnp.float32),
        pltpu.VMEM((CH, DP), jnp.float32),
        pltpu.VMEM((BPW,), jnp.float32),
        pltpu.SemaphoreType.DMA,
    ],
)(_sc_body)


def kernel(network, src, trg, emb):
    full_cols = (NBLK - 1) * IB

    def packed_row(x):
        blk = x >> IBL
        j_main = ((blk >> 1) << IBL) | (x & (IB - 1))
        j_tail = (NPAIR - 1) * IB + (x - full_cols)
        return jnp.where(x < full_cols, j_main, j_tail)

    def lane_off(x):
        return jnp.where(x < full_cols, ((x >> IBL) & 1) << 6, 0)

    src32 = src.astype(jnp.int32)
    trg32 = trg.astype(jnp.int32)
    srch = packed_row(src32).reshape(NW * NCH, CH)
    trgh = packed_row(trg32).reshape(NW * NCH, CH)
    spar = lane_off(src32).reshape(NW * NCH, CH)
    tpar = lane_off(trg32).reshape(NW * NCH, CH)
    packed = _detile(emb.T)
    return _sc_kernel(srch, trgh, spar, tpar, packed)
